# baseline (speedup 1.0000x reference)
import functools
import jax
import jax.numpy as jnp
from jax import lax
from jax.experimental import pallas as pl
from jax.experimental.pallas import tpu as pltpu
from jax.experimental.pallas import tpu_sc as plsc

N = 10000
NPAD = 10240
E = 320000
LOOKBACK = 7
F = 5
H = 128
OUT = 5
XW = 48

NSC = 2
NTILE = 16
ROWS_PER_TILE = 80
EPT = ROWS_PER_TILE * 128
EPAD = EPT * NSC * NTILE
ROWS_TOTAL = EPAD // 128
NSLICE = NPAD // NTILE

ROWS_C0 = 80
ROWS_C1 = 80


def _deg_body(dst_hbm, z16_hbm, out_hbm, idx_v, ones_v, acc_sh):
    cid = lax.axis_index("c")
    sid = lax.axis_index("s")

    def fill(i, carry):
        ones_v[i, :] = jnp.ones((16,), jnp.float32)
        return carry
    lax.fori_loop(0, 128, fill, 0)

    row0 = sid * NSLICE
    pltpu.sync_copy(z16_hbm.at[pl.ds(row0, NSLICE)],
                    acc_sh.at[pl.ds(row0, NSLICE)])
    base_row = (cid * NTILE + sid) * ROWS_PER_TILE
    pltpu.sync_copy(dst_hbm.at[pl.ds(base_row, ROWS_PER_TILE)], idx_v)
    plsc.subcore_barrier()

    def step(j, carry):
        pltpu.sync_copy(ones_v, acc_sh.at[idx_v.at[j]], add=True)
        return carry
    lax.fori_loop(0, ROWS_PER_TILE, step, 0)

    plsc.subcore_barrier()
    pltpu.sync_copy(acc_sh.at[pl.ds(row0, NSLICE)],
                    out_hbm.at[pl.ds(cid * NPAD + row0, NSLICE)])


def _fill_zeros(zbuf, width):
    def fill(i, carry):
        r = i // (width // 16)
        c = (i % (width // 16)) * 16
        zbuf[r, pl.ds(c, 16)] = jnp.zeros((16,), jnp.float32)
        return carry
    lax.fori_loop(0, 8 * (width // 16), fill, 0)


def _zero_acc_slice(zbuf, acc_sh, row0, semz):
    def fire(j, carry):
        pltpu.async_copy(zbuf, acc_sh.at[pl.ds(row0 + j * 8, 8)], semz)
        return carry
    lax.fori_loop(0, NSLICE // 8, fire, 0)

    def drain(j, carry):
        pltpu.make_async_copy(
            zbuf, acc_sh.at[pl.ds(row0 + j * 8, 8)], semz).wait()
        return carry
    lax.fori_loop(0, NSLICE // 8, drain, 0)


def _gather_scatter_rows(table, srcv, dstv, bufa, bufb, acc_sh,
                         sema, semb, nrows):
    def step(j2, carry):
        r0 = 2 * j2
        r1 = r0 + 1
        h0 = pltpu.async_copy(table.at[srcv.at[r0]], bufa, sema)
        h1 = pltpu.async_copy(table.at[srcv.at[r1]], bufb, semb)
        h0.wait()
        pltpu.sync_copy(bufa, acc_sh.at[dstv.at[r0]], add=True)
        h1.wait()
        pltpu.sync_copy(bufb, acc_sh.at[dstv.at[r1]], add=True)
        return carry
    lax.fori_loop(0, nrows // 2, step, 0)


def _pass1_body(x48_hbm, src_hbm, dst_hbm, out_hbm,
              srcv, dstv, bufa, bufb, zbuf, acc_sh, sema, semb):
    cid = lax.axis_index("c")
    sid = lax.axis_index("s")
    row0 = sid * NSLICE
    _fill_zeros(zbuf, XW)
    _zero_acc_slice(zbuf, acc_sh, row0, sema)
    base_row = (cid * NTILE + sid) * ROWS_PER_TILE
    pltpu.sync_copy(src_hbm.at[pl.ds(base_row, ROWS_PER_TILE)], srcv)
    pltpu.sync_copy(dst_hbm.at[pl.ds(base_row, ROWS_PER_TILE)], dstv)
    plsc.subcore_barrier()

    _gather_scatter_rows(x48_hbm, srcv, dstv, bufa, bufb, acc_sh, sema, semb,
                         ROWS_PER_TILE)

    plsc.subcore_barrier()
    pltpu.sync_copy(acc_sh.at[pl.ds(row0, NSLICE)],
                    out_hbm.at[pl.ds(cid * NPAD + row0, NSLICE)])


def _pass2_body(g0, g1, g2, g3, g4, g5, g6, src_hbm, dst_hbm,
              o0, o1, o2, o3, o4, o5, o6,
              srcv, dstv, bufa, bufb, zbuf, acc_sh, sema, semb):
    cid = lax.axis_index("c")
    sid = lax.axis_index("s")
    row0 = sid * NSLICE
    _fill_zeros(zbuf, H)

    tables = (g0, g1, g2, g3, g4, g5, g6)
    outs = (o0, o1, o2, o3, o4, o5, o6)
    for k in range(LOOKBACK):
        _zero_acc_slice(zbuf, acc_sh, row0, sema)
        plsc.subcore_barrier()

        for core, rows, base0 in ((0, ROWS_C0, 0),
                                  (1, ROWS_C1, ROWS_C0 * NTILE)):
            @pl.when(cid == core)
            def _(rows=rows, base0=base0):
                base_row = base0 + sid * rows
                half = rows // 2
                for ph in range(2):
                    pltpu.sync_copy(
                        src_hbm.at[pl.ds(base_row + ph * half, half)],
                        srcv.at[pl.ds(0, half)])
                    pltpu.sync_copy(
                        dst_hbm.at[pl.ds(base_row + ph * half, half)],
                        dstv.at[pl.ds(0, half)])
                    _gather_scatter_rows(tables[k], srcv, dstv, bufa, bufb,
                                         acc_sh, sema, semb, half)

        plsc.subcore_barrier()
        pltpu.sync_copy(acc_sh.at[pl.ds(row0, NSLICE)],
                        outs[k].at[pl.ds(cid * NPAD + row0, NSLICE)])


@functools.lru_cache(maxsize=1)
def _sc_kernels():
    mesh = plsc.VectorSubcoreMesh(core_axis_name="c", subcore_axis_name="s")
    deg = pl.kernel(
        _deg_body,
        mesh=mesh,
        out_type=jax.ShapeDtypeStruct((NSC * NPAD, 16), jnp.float32),
        scratch_types=[
            pltpu.VMEM((ROWS_PER_TILE, 128), jnp.int32),
            pltpu.VMEM((128, 16), jnp.float32),
            pltpu.VMEM_SHARED((NPAD, 16), jnp.float32),
        ],
    )
    pass1 = pl.kernel(
        _pass1_body,
        mesh=mesh,
        compiler_params=pltpu.CompilerParams(use_tc_tiling_on_sc=False),
        out_type=jax.ShapeDtypeStruct((NSC * NPAD, XW), jnp.float32),
        scratch_types=[
            pltpu.VMEM((ROWS_PER_TILE, 128), jnp.int32),
            pltpu.VMEM((ROWS_PER_TILE, 128), jnp.int32),
            pltpu.VMEM((128, XW), jnp.float32),
            pltpu.VMEM((128, XW), jnp.float32),
            pltpu.VMEM((8, XW), jnp.float32),
            pltpu.VMEM_SHARED((NPAD, XW), jnp.float32),
            pltpu.SemaphoreType.DMA,
            pltpu.SemaphoreType.DMA,
        ],
    )
    pass2 = pl.kernel(
        _pass2_body,
        mesh=mesh,
        out_type=[jax.ShapeDtypeStruct((NSC * NPAD, H), jnp.float32)
                  for _ in range(LOOKBACK)],
        scratch_types=[
            pltpu.VMEM((max(ROWS_C0, ROWS_C1) // 2, 128), jnp.int32),
            pltpu.VMEM((max(ROWS_C0, ROWS_C1) // 2, 128), jnp.int32),
            pltpu.VMEM((128, H), jnp.float32),
            pltpu.VMEM((128, H), jnp.float32),
            pltpu.VMEM((8, H), jnp.float32),
            pltpu.VMEM_SHARED((NPAD, H), jnp.float32),
            pltpu.SemaphoreType.DMA,
            pltpu.SemaphoreType.DMA,
        ],
    )
    return deg, pass1, pass2


def _sc_deg(dstp, z16):
    return _sc_kernels()[0](dstp, z16)


def _sc_pass1(x48, srcp, dstp):
    return _sc_kernels()[1](x48, srcp, dstp)


def _sc_pass2(*args):
    return _sc_kernels()[2](*args)


_BNP = 2048
_BN1 = 1024
_BN2 = 512


def _prep_body(degp_ref, xp_ref, x48_ref, dinv_ref):
    deg = 1.0 + degp_ref[0, :, 0:1] + degp_ref[1, :, 0:1]
    dv = lax.rsqrt(deg)
    x48_ref[...] = xp_ref[...] * dv
    dinv_ref[...] = jnp.broadcast_to(dv, dinv_ref.shape)


def _prep_call(degp, xp48):
    return pl.pallas_call(
        _prep_body,
        grid=(NPAD // _BNP,),
        in_specs=[
            pl.BlockSpec((NSC, _BNP, 16), lambda i: (0, i, 0)),
            pl.BlockSpec((_BNP, XW), lambda i: (i, 0)),
        ],
        out_specs=[
            pl.BlockSpec((_BNP, XW), lambda i: (i, 0)),
            pl.BlockSpec((_BNP, H), lambda i: (i, 0)),
        ],
        out_shape=[
            jax.ShapeDtypeStruct((NPAD, XW), jnp.float32),
            jax.ShapeDtypeStruct((NPAD, H), jnp.float32),
        ],
    )(degp, xp48)


def _conv1_body(s1p_ref, x48_ref, dinv_ref, w1_ref, b1_ref, *out_refs):
    dv = dinv_ref[...]
    s1 = (s1p_ref[0] + s1p_ref[1] + x48_ref[...]) * dv[:, 0:1]
    z = jnp.dot(s1, w1_ref[...], preferred_element_type=jnp.float32)
    z = z + b1_ref[...]
    for k in range(LOOKBACK):
        out_refs[k][...] = jnp.maximum(z[:, k * H:(k + 1) * H], 0.0) * dv


def _conv1_call(s1p, x48, dinv, w1big, b1t):
    return pl.pallas_call(
        _conv1_body,
        grid=(NPAD // _BN1,),
        in_specs=[
            pl.BlockSpec((NSC, _BN1, XW), lambda i: (0, i, 0)),
            pl.BlockSpec((_BN1, XW), lambda i: (i, 0)),
            pl.BlockSpec((_BN1, H), lambda i: (i, 0)),
            pl.BlockSpec((XW, LOOKBACK * H), lambda i: (0, 0)),
            pl.BlockSpec((1, LOOKBACK * H), lambda i: (0, 0)),
        ],
        out_specs=[pl.BlockSpec((_BN1, H), lambda i: (i, 0))
                   for _ in range(LOOKBACK)],
        out_shape=[jax.ShapeDtypeStruct((NPAD, H), jnp.float32)
                   for _ in range(LOOKBACK)],
    )(s1p, x48, dinv, w1big, b1t)


def _final_body(*refs):
    s2p = refs[0:LOOKBACK]
    g1 = refs[LOOKBACK:2 * LOOKBACK]
    (dinv_ref, w2_ref, b2_ref, wih_ref, whh_ref, bih_ref, bhh_ref,
     wout_ref, bout_ref, out_ref) = refs[2 * LOOKBACK:]
    dv = dinv_ref[...]
    w2 = w2_ref[...]
    wih = wih_ref[...]
    whh = whh_ref[...]
    h = jnp.zeros((dv.shape[0], H), jnp.float32)
    for k in range(LOOKBACK):
        t = (s2p[k][0] + s2p[k][1] + g1[k][...]) * dv
        g = jnp.dot(t, w2, preferred_element_type=jnp.float32) + b2_ref[...]
        g = jnp.maximum(g, 0.0)
        gi = jnp.dot(g, wih, preferred_element_type=jnp.float32) + bih_ref[...]
        gh = jnp.dot(h, whh, preferred_element_type=jnp.float32) + bhh_ref[...]
        r = jax.nn.sigmoid(gi[:, :H] + gh[:, :H])
        z = jax.nn.sigmoid(gi[:, H:2 * H] + gh[:, H:2 * H])
        n = jnp.tanh(gi[:, 2 * H:] + r * gh[:, 2 * H:])
        h = (1.0 - z) * n + z * h
    logits = jnp.dot(h, wout_ref[...], preferred_element_type=jnp.float32)
    logits = logits + bout_ref[...]
    m = jnp.max(logits, axis=1, keepdims=True)
    e = jnp.exp(logits - m)
    out_ref[...] = e / jnp.sum(e, axis=1, keepdims=True)


def _final_call(s2, g1, dinv, w2, b2, wihT, whhT, bih, bhh, wout8, bout8):
    n_in = 2 * LOOKBACK + 9
    specs = ([pl.BlockSpec((NSC, _BN2, H), lambda i: (0, i, 0))
              for _ in range(LOOKBACK)] +
             [pl.BlockSpec((_BN2, H), lambda i: (i, 0))
              for _ in range(LOOKBACK)] +
             [pl.BlockSpec((_BN2, H), lambda i: (i, 0)),
              pl.BlockSpec((H, H), lambda i: (0, 0)),
              pl.BlockSpec((1, H), lambda i: (0, 0)),
              pl.BlockSpec((H, 3 * H), lambda i: (0, 0)),
              pl.BlockSpec((H, 3 * H), lambda i: (0, 0)),
              pl.BlockSpec((1, 3 * H), lambda i: (0, 0)),
              pl.BlockSpec((1, 3 * H), lambda i: (0, 0)),
              pl.BlockSpec((H, 8), lambda i: (0, 0)),
              pl.BlockSpec((1, 8), lambda i: (0, 0))])
    assert len(specs) == n_in
    return pl.pallas_call(
        _final_body,
        grid=(NPAD // _BN2,),
        in_specs=specs,
        out_specs=pl.BlockSpec((_BN2, 8), lambda i: (i, 0)),
        out_shape=jax.ShapeDtypeStruct((NPAD, 8), jnp.float32),
    )(*s2, *g1, dinv, w2, b2, wihT, whhT, bih, bhh, wout8, bout8)


def kernel(x, edge_index, batch, W1, b1, W2, b2, W_ih, W_hh, b_ih, b_hh,
           W_out, b_out):
    pad = EPAD - E
    srcp = jnp.concatenate(
        [edge_index[0], jnp.zeros((pad,), jnp.int32)]).reshape(ROWS_TOTAL, 128)
    dstp = jnp.concatenate(
        [edge_index[1], jnp.full((pad,), N, jnp.int32)]).reshape(ROWS_TOTAL, 128)
    z16 = jnp.zeros((NPAD, 16), jnp.float32)
    xp48 = jnp.zeros((NPAD, XW), jnp.float32).at[:N, :LOOKBACK * F].set(
        x.reshape(N, LOOKBACK * F))

    w1big = jnp.zeros((XW, LOOKBACK * H), jnp.float32)
    for k in range(LOOKBACK):
        w1big = w1big.at[k * F:(k + 1) * F, k * H:(k + 1) * H].set(W1)
    b1t = jnp.tile(b1, LOOKBACK).reshape(1, LOOKBACK * H)

    wout8 = jnp.zeros((H, 8), jnp.float32).at[:, :OUT].set(W_out)
    bout8 = jnp.full((1, 8), -1e30, jnp.float32).at[0, :OUT].set(b_out)

    degp = _sc_deg(dstp, z16).reshape(NSC, NPAD, 16)
    x48, dinv = _prep_call(degp, xp48)
    s1p = _sc_pass1(x48, srcp, dstp).reshape(NSC, NPAD, XW)
    g1 = _conv1_call(s1p, x48, dinv, w1big, b1t)
    s2flat = _sc_pass2(*g1, srcp, dstp)
    s2 = [s.reshape(NSC, NPAD, H) for s in s2flat]
    out8 = _final_call(s2, g1, dinv, W2, b2.reshape(1, H), W_ih.T, W_hh.T,
                       b_ih.reshape(1, 3 * H), b_hh.reshape(1, 3 * H),
                       wout8, bout8)
    return out8[:N, :OUT]


# speedup vs baseline: 1.1060x; 1.1060x over previous
import functools
import jax
import jax.numpy as jnp
from jax import lax
from jax.experimental import pallas as pl
from jax.experimental.pallas import tpu as pltpu
from jax.experimental.pallas import tpu_sc as plsc

N = 10000
NPAD = 10240
E = 320000
LOOKBACK = 7
F = 5
H = 128
OUT = 5
XW = 48

NSC = 2
NTILE = 16
ROWS_PER_TILE = 80
EPT = ROWS_PER_TILE * 128
EPAD = EPT * NSC * NTILE
ROWS_TOTAL = EPAD // 128
NSLICE = NPAD // NTILE

ROWS_C0 = 112
ROWS_C1 = 48


def _deg_body(dst_hbm, z16_hbm, out_hbm, idx_v, ones_v, acc_sh):
    cid = lax.axis_index("c")
    sid = lax.axis_index("s")

    def fill(i, carry):
        ones_v[i, :] = jnp.ones((16,), jnp.float32)
        return carry
    lax.fori_loop(0, 128, fill, 0)

    row0 = sid * NSLICE
    pltpu.sync_copy(z16_hbm.at[pl.ds(row0, NSLICE)],
                    acc_sh.at[pl.ds(row0, NSLICE)])
    base_row = (cid * NTILE + sid) * ROWS_PER_TILE
    pltpu.sync_copy(dst_hbm.at[pl.ds(base_row, ROWS_PER_TILE)], idx_v)
    plsc.subcore_barrier()

    def step(j, carry):
        pltpu.sync_copy(ones_v, acc_sh.at[idx_v.at[j]], add=True)
        return carry
    lax.fori_loop(0, ROWS_PER_TILE, step, 0)

    plsc.subcore_barrier()
    pltpu.sync_copy(acc_sh.at[pl.ds(row0, NSLICE)],
                    out_hbm.at[pl.ds(cid * NPAD + row0, NSLICE)])


def _fill_zeros(zbuf, width):
    def fill(i, carry):
        r = i // (width // 16)
        c = (i % (width // 16)) * 16
        zbuf[r, pl.ds(c, 16)] = jnp.zeros((16,), jnp.float32)
        return carry
    lax.fori_loop(0, 8 * (width // 16), fill, 0)


def _zero_acc_slice(zbuf, acc_sh, row0, semz):
    def fire(j, carry):
        pltpu.async_copy(zbuf, acc_sh.at[pl.ds(row0 + j * 8, 8)], semz)
        return carry
    lax.fori_loop(0, NSLICE // 8, fire, 0)

    def drain(j, carry):
        pltpu.make_async_copy(
            zbuf, acc_sh.at[pl.ds(row0 + j * 8, 8)], semz).wait()
        return carry
    lax.fori_loop(0, NSLICE // 8, drain, 0)


def _gather_scatter_rows(table, srcv, dstv, bufa, bufb, acc_sh,
                         sema, semb, nrows):
    def step(j2, carry):
        r0 = 2 * j2
        r1 = r0 + 1
        h0 = pltpu.async_copy(table.at[srcv.at[r0]], bufa, sema)
        h1 = pltpu.async_copy(table.at[srcv.at[r1]], bufb, semb)
        h0.wait()
        pltpu.sync_copy(bufa, acc_sh.at[dstv.at[r0]], add=True)
        h1.wait()
        pltpu.sync_copy(bufb, acc_sh.at[dstv.at[r1]], add=True)
        return carry
    lax.fori_loop(0, nrows // 2, step, 0)


def _pass1_body(x48_hbm, src_hbm, dst_hbm, out_hbm,
              srcv, dstv, bufa, bufb, zbuf, acc_sh, sema, semb):
    cid = lax.axis_index("c")
    sid = lax.axis_index("s")
    row0 = sid * NSLICE
    _fill_zeros(zbuf, XW)
    _zero_acc_slice(zbuf, acc_sh, row0, sema)
    base_row = (cid * NTILE + sid) * ROWS_PER_TILE
    pltpu.sync_copy(src_hbm.at[pl.ds(base_row, ROWS_PER_TILE)], srcv)
    pltpu.sync_copy(dst_hbm.at[pl.ds(base_row, ROWS_PER_TILE)], dstv)
    plsc.subcore_barrier()

    _gather_scatter_rows(x48_hbm, srcv, dstv, bufa, bufb, acc_sh, sema, semb,
                         ROWS_PER_TILE)

    plsc.subcore_barrier()
    pltpu.sync_copy(acc_sh.at[pl.ds(row0, NSLICE)],
                    out_hbm.at[pl.ds(cid * NPAD + row0, NSLICE)])


def _pass2_body(g0, g1, g2, g3, g4, g5, g6, src_hbm, dst_hbm,
              o0, o1, o2, o3, o4, o5, o6,
              srcv, dstv, bufa, bufb, zbuf, acc_sh, sema, semb):
    cid = lax.axis_index("c")
    sid = lax.axis_index("s")
    row0 = sid * NSLICE
    _fill_zeros(zbuf, H)

    tables = (g0, g1, g2, g3, g4, g5, g6)
    outs = (o0, o1, o2, o3, o4, o5, o6)
    for k in range(LOOKBACK):
        _zero_acc_slice(zbuf, acc_sh, row0, sema)
        plsc.subcore_barrier()

        for core, rows, base0 in ((0, ROWS_C0, 0),
                                  (1, ROWS_C1, ROWS_C0 * NTILE)):
            @pl.when(cid == core)
            def _(rows=rows, base0=base0):
                base_row = base0 + sid * rows
                half = rows // 2
                for ph in range(2):
                    pltpu.sync_copy(
                        src_hbm.at[pl.ds(base_row + ph * half, half)],
                        srcv.at[pl.ds(0, half)])
                    pltpu.sync_copy(
                        dst_hbm.at[pl.ds(base_row + ph * half, half)],
                        dstv.at[pl.ds(0, half)])
                    _gather_scatter_rows(tables[k], srcv, dstv, bufa, bufb,
                                         acc_sh, sema, semb, half)

        plsc.subcore_barrier()
        pltpu.sync_copy(acc_sh.at[pl.ds(row0, NSLICE)],
                        outs[k].at[pl.ds(cid * NPAD + row0, NSLICE)])


@functools.lru_cache(maxsize=1)
def _sc_kernels():
    mesh = plsc.VectorSubcoreMesh(core_axis_name="c", subcore_axis_name="s")
    deg = pl.kernel(
        _deg_body,
        mesh=mesh,
        out_type=jax.ShapeDtypeStruct((NSC * NPAD, 16), jnp.float32),
        scratch_types=[
            pltpu.VMEM((ROWS_PER_TILE, 128), jnp.int32),
            pltpu.VMEM((128, 16), jnp.float32),
            pltpu.VMEM_SHARED((NPAD, 16), jnp.float32),
        ],
    )
    pass1 = pl.kernel(
        _pass1_body,
        mesh=mesh,
        compiler_params=pltpu.CompilerParams(use_tc_tiling_on_sc=False),
        out_type=jax.ShapeDtypeStruct((NSC * NPAD, XW), jnp.float32),
        scratch_types=[
            pltpu.VMEM((ROWS_PER_TILE, 128), jnp.int32),
            pltpu.VMEM((ROWS_PER_TILE, 128), jnp.int32),
            pltpu.VMEM((128, XW), jnp.float32),
            pltpu.VMEM((128, XW), jnp.float32),
            pltpu.VMEM((8, XW), jnp.float32),
            pltpu.VMEM_SHARED((NPAD, XW), jnp.float32),
            pltpu.SemaphoreType.DMA,
            pltpu.SemaphoreType.DMA,
        ],
    )
    pass2 = pl.kernel(
        _pass2_body,
        mesh=mesh,
        out_type=[jax.ShapeDtypeStruct((NSC * NPAD, H), jnp.float32)
                  for _ in range(LOOKBACK)],
        scratch_types=[
            pltpu.VMEM((max(ROWS_C0, ROWS_C1) // 2, 128), jnp.int32),
            pltpu.VMEM((max(ROWS_C0, ROWS_C1) // 2, 128), jnp.int32),
            pltpu.VMEM((128, H), jnp.float32),
            pltpu.VMEM((128, H), jnp.float32),
            pltpu.VMEM((8, H), jnp.float32),
            pltpu.VMEM_SHARED((NPAD, H), jnp.float32),
            pltpu.SemaphoreType.DMA,
            pltpu.SemaphoreType.DMA,
        ],
    )
    return deg, pass1, pass2


def _sc_deg(dstp, z16):
    return _sc_kernels()[0](dstp, z16)


def _sc_pass1(x48, srcp, dstp):
    return _sc_kernels()[1](x48, srcp, dstp)


def _sc_pass2(*args):
    return _sc_kernels()[2](*args)


_BNP = 2048
_BN1 = 1024
_BN2 = 512


def _prep_body(degp_ref, xp_ref, x48_ref, dinv_ref):
    deg = 1.0 + degp_ref[0, :, 0:1] + degp_ref[1, :, 0:1]
    dv = lax.rsqrt(deg)
    x48_ref[...] = xp_ref[...] * dv
    dinv_ref[...] = jnp.broadcast_to(dv, dinv_ref.shape)


def _prep_call(degp, xp48):
    return pl.pallas_call(
        _prep_body,
        grid=(NPAD // _BNP,),
        in_specs=[
            pl.BlockSpec((NSC, _BNP, 16), lambda i: (0, i, 0)),
            pl.BlockSpec((_BNP, XW), lambda i: (i, 0)),
        ],
        out_specs=[
            pl.BlockSpec((_BNP, XW), lambda i: (i, 0)),
            pl.BlockSpec((_BNP, H), lambda i: (i, 0)),
        ],
        out_shape=[
            jax.ShapeDtypeStruct((NPAD, XW), jnp.float32),
            jax.ShapeDtypeStruct((NPAD, H), jnp.float32),
        ],
    )(degp, xp48)


def _conv1_body(s1p_ref, x48_ref, dinv_ref, w1_ref, b1_ref, *out_refs):
    dv = dinv_ref[...]
    s1 = (s1p_ref[0] + s1p_ref[1] + x48_ref[...]) * dv[:, 0:1]
    z = jnp.dot(s1, w1_ref[...], preferred_element_type=jnp.float32)
    z = z + b1_ref[...]
    for k in range(LOOKBACK):
        out_refs[k][...] = jnp.maximum(z[:, k * H:(k + 1) * H], 0.0) * dv


def _conv1_call(s1p, x48, dinv, w1big, b1t):
    return pl.pallas_call(
        _conv1_body,
        grid=(NPAD // _BN1,),
        in_specs=[
            pl.BlockSpec((NSC, _BN1, XW), lambda i: (0, i, 0)),
            pl.BlockSpec((_BN1, XW), lambda i: (i, 0)),
            pl.BlockSpec((_BN1, H), lambda i: (i, 0)),
            pl.BlockSpec((XW, LOOKBACK * H), lambda i: (0, 0)),
            pl.BlockSpec((1, LOOKBACK * H), lambda i: (0, 0)),
        ],
        out_specs=[pl.BlockSpec((_BN1, H), lambda i: (i, 0))
                   for _ in range(LOOKBACK)],
        out_shape=[jax.ShapeDtypeStruct((NPAD, H), jnp.float32)
                   for _ in range(LOOKBACK)],
    )(s1p, x48, dinv, w1big, b1t)


def _final_body(*refs):
    s2p = refs[0:LOOKBACK]
    g1 = refs[LOOKBACK:2 * LOOKBACK]
    (dinv_ref, w2_ref, b2_ref, wih_ref, whh_ref, bih_ref, bhh_ref,
     wout_ref, bout_ref, out_ref) = refs[2 * LOOKBACK:]
    dv = dinv_ref[...]
    w2 = w2_ref[...]
    wih = wih_ref[...]
    whh = whh_ref[...]
    h = jnp.zeros((dv.shape[0], H), jnp.float32)
    for k in range(LOOKBACK):
        t = (s2p[k][0] + s2p[k][1] + g1[k][...]) * dv
        g = jnp.dot(t, w2, preferred_element_type=jnp.float32) + b2_ref[...]
        g = jnp.maximum(g, 0.0)
        gi = jnp.dot(g, wih, preferred_element_type=jnp.float32) + bih_ref[...]
        gh = jnp.dot(h, whh, preferred_element_type=jnp.float32) + bhh_ref[...]
        r = jax.nn.sigmoid(gi[:, :H] + gh[:, :H])
        z = jax.nn.sigmoid(gi[:, H:2 * H] + gh[:, H:2 * H])
        n = jnp.tanh(gi[:, 2 * H:] + r * gh[:, 2 * H:])
        h = (1.0 - z) * n + z * h
    logits = jnp.dot(h, wout_ref[...], preferred_element_type=jnp.float32)
    logits = logits + bout_ref[...]
    m = jnp.max(logits, axis=1, keepdims=True)
    e = jnp.exp(logits - m)
    out_ref[...] = e / jnp.sum(e, axis=1, keepdims=True)


def _final_call(s2, g1, dinv, w2, b2, wihT, whhT, bih, bhh, wout8, bout8):
    n_in = 2 * LOOKBACK + 9
    specs = ([pl.BlockSpec((NSC, _BN2, H), lambda i: (0, i, 0))
              for _ in range(LOOKBACK)] +
             [pl.BlockSpec((_BN2, H), lambda i: (i, 0))
              for _ in range(LOOKBACK)] +
             [pl.BlockSpec((_BN2, H), lambda i: (i, 0)),
              pl.BlockSpec((H, H), lambda i: (0, 0)),
              pl.BlockSpec((1, H), lambda i: (0, 0)),
              pl.BlockSpec((H, 3 * H), lambda i: (0, 0)),
              pl.BlockSpec((H, 3 * H), lambda i: (0, 0)),
              pl.BlockSpec((1, 3 * H), lambda i: (0, 0)),
              pl.BlockSpec((1, 3 * H), lambda i: (0, 0)),
              pl.BlockSpec((H, 8), lambda i: (0, 0)),
              pl.BlockSpec((1, 8), lambda i: (0, 0))])
    assert len(specs) == n_in
    return pl.pallas_call(
        _final_body,
        grid=(NPAD // _BN2,),
        in_specs=specs,
        out_specs=pl.BlockSpec((_BN2, 8), lambda i: (i, 0)),
        out_shape=jax.ShapeDtypeStruct((NPAD, 8), jnp.float32),
    )(*s2, *g1, dinv, w2, b2, wihT, whhT, bih, bhh, wout8, bout8)


def kernel(x, edge_index, batch, W1, b1, W2, b2, W_ih, W_hh, b_ih, b_hh,
           W_out, b_out):
    pad = EPAD - E
    srcp = jnp.concatenate(
        [edge_index[0], jnp.zeros((pad,), jnp.int32)]).reshape(ROWS_TOTAL, 128)
    dstp = jnp.concatenate(
        [edge_index[1], jnp.full((pad,), N, jnp.int32)]).reshape(ROWS_TOTAL, 128)
    z16 = jnp.zeros((NPAD, 16), jnp.float32)
    xp48 = jnp.zeros((NPAD, XW), jnp.float32).at[:N, :LOOKBACK * F].set(
        x.reshape(N, LOOKBACK * F))

    w1big = jnp.zeros((XW, LOOKBACK * H), jnp.float32)
    for k in range(LOOKBACK):
        w1big = w1big.at[k * F:(k + 1) * F, k * H:(k + 1) * H].set(W1)
    b1t = jnp.tile(b1, LOOKBACK).reshape(1, LOOKBACK * H)

    wout8 = jnp.zeros((H, 8), jnp.float32).at[:, :OUT].set(W_out)
    bout8 = jnp.full((1, 8), -1e30, jnp.float32).at[0, :OUT].set(b_out)

    degp = _sc_deg(dstp, z16).reshape(NSC, NPAD, 16)
    x48, dinv = _prep_call(degp, xp48)
    s1p = _sc_pass1(x48, srcp, dstp).reshape(NSC, NPAD, XW)
    g1 = _conv1_call(s1p, x48, dinv, w1big, b1t)
    s2flat = _sc_pass2(*g1, srcp, dstp)
    s2 = [s.reshape(NSC, NPAD, H) for s in s2flat]
    out8 = _final_call(s2, g1, dinv, W2, b2.reshape(1, H), W_ih.T, W_hh.T,
                       b_ih.reshape(1, 3 * H), b_hh.reshape(1, 3 * H),
                       wout8, bout8)
    return out8[:N, :OUT]
